# Initial kernel scaffold; baseline (speedup 1.0000x reference)
import jax
import jax.numpy as jnp
from jax.experimental import pallas as pl


def kernel(node_feats, edge_feats, edge_index, W1, b1, W2, b2, U1, bu1, U2, bu2):
    raise NotImplementedError("write your pallas kernel here")


# speedup vs baseline: 3.3333x; 3.3333x over previous
import functools

import jax
import jax.numpy as jnp
from jax import lax
from jax.experimental import pallas as pl
from jax.experimental.pallas import tpu as pltpu
from jax.experimental.pallas import tpu_sc as plsc

N_NODES = 10000
NODE_DIM = 128
HIDDEN = 128
EDGE_DIM = 16
N_EDGES = 320000

NC, NS = 2, 16
NW = NC * NS
CHUNK = 64
NCHUNKS = N_EDGES // CHUNK
BASE_CHUNKS = NCHUNKS // NW
EXTRA = NCHUNKS - BASE_CHUNKS * NW
SUB_ROWS = 624
TAIL_ROWS = N_NODES - NS * SUB_ROWS
DEG_W = 16
LANES = 16


PREP_GRID = 125
E_BLK = N_EDGES // PREP_GRID
N_BLK = N_NODES // PREP_GRID


def _prep_body(nf_ref, ef_ref, w1s_ref, w1t_ref, w1e_ref, b1_ref,
               a_ref, b_ref, c_ref):
    nf = nf_ref[...]
    a_ref[...] = jnp.dot(nf, w1s_ref[...], preferred_element_type=jnp.float32)
    b_ref[...] = jnp.dot(nf, w1t_ref[...], preferred_element_type=jnp.float32)
    c_ref[...] = (jnp.dot(ef_ref[...], w1e_ref[...],
                          preferred_element_type=jnp.float32) + b1_ref[...])


_prep = pl.pallas_call(
    _prep_body,
    grid=(PREP_GRID,),
    in_specs=[
        pl.BlockSpec((N_BLK, NODE_DIM), lambda i: (i, 0)),
        pl.BlockSpec((E_BLK, EDGE_DIM), lambda i: (i, 0)),
        pl.BlockSpec((NODE_DIM, HIDDEN), lambda i: (0, 0)),
        pl.BlockSpec((NODE_DIM, HIDDEN), lambda i: (0, 0)),
        pl.BlockSpec((EDGE_DIM, HIDDEN), lambda i: (0, 0)),
        pl.BlockSpec((1, HIDDEN), lambda i: (0, 0)),
    ],
    out_specs=[
        pl.BlockSpec((N_BLK, HIDDEN), lambda i: (i, 0)),
        pl.BlockSpec((N_BLK, HIDDEN), lambda i: (i, 0)),
        pl.BlockSpec((E_BLK, HIDDEN), lambda i: (i, 0)),
    ],
    out_shape=[
        jax.ShapeDtypeStruct((N_NODES, HIDDEN), jnp.float32),
        jax.ShapeDtypeStruct((N_NODES, HIDDEN), jnp.float32),
        jax.ShapeDtypeStruct((N_EDGES, HIDDEN), jnp.float32),
    ],
)


_sc_mesh = plsc.VectorSubcoreMesh(core_axis_name="c", subcore_axis_name="s")


@functools.partial(
    pl.kernel,
    mesh=_sc_mesh,
    compiler_params=pltpu.CompilerParams(use_tc_tiling_on_sc=False),
    out_type=[
        pltpu.HBM((NC, N_NODES, HIDDEN), jnp.float32),
        pltpu.HBM((NC, N_NODES, DEG_W), jnp.float32),
    ],
    scratch_types=[
        pltpu.VMEM((CHUNK,), jnp.int32),
        pltpu.VMEM((CHUNK,), jnp.int32),
        pltpu.VMEM((CHUNK, HIDDEN), jnp.float32),
        pltpu.VMEM((CHUNK, HIDDEN), jnp.float32),
        pltpu.VMEM((CHUNK, HIDDEN), jnp.float32),
        pltpu.VMEM((CHUNK, DEG_W), jnp.float32),
        pltpu.VMEM_SHARED((N_NODES, HIDDEN), jnp.float32),
        pltpu.VMEM_SHARED((N_NODES, DEG_W), jnp.float32),
        pltpu.SemaphoreType.DMA,
        pltpu.SemaphoreType.DMA,
    ],
)
def _sc_aggregate(a_hbm, b_hbm, c_hbm, src_hbm, tgt_hbm,
                  hsum_out, deg_out,
                  src_v, tgt_v, arows, brows, crows, degrow,
                  hsum_sh, deg_sh, sem1, sem2):
    cid = lax.axis_index("c")
    sid = lax.axis_index("s")
    wid = sid * NC + cid

    lanes = lax.iota(jnp.int32, LANES)
    zero16 = jnp.zeros((LANES,), jnp.float32)
    one0 = jnp.where(lanes == 0, jnp.float32(1.0), jnp.float32(0.0))

    def _fill_zeros(r, _):
        for cc in range(HIDDEN // LANES):
            arows[r, pl.ds(cc * LANES, LANES)] = zero16
        degrow[r, pl.ds(0, LANES)] = zero16
        return 0

    lax.fori_loop(0, CHUNK, _fill_zeros, 0)

    row0 = sid * SUB_ROWS
    for z in range(SUB_ROWS // CHUNK):
        pltpu.sync_copy(arows, hsum_sh.at[pl.ds(row0 + z * CHUNK, CHUNK)])
        pltpu.sync_copy(degrow, deg_sh.at[pl.ds(row0 + z * CHUNK, CHUNK)])
    _rem = SUB_ROWS - (SUB_ROWS // CHUNK) * CHUNK
    pltpu.sync_copy(arows.at[pl.ds(0, _rem)],
                    hsum_sh.at[pl.ds(row0 + SUB_ROWS - _rem, _rem)])
    pltpu.sync_copy(degrow.at[pl.ds(0, _rem)],
                    deg_sh.at[pl.ds(row0 + SUB_ROWS - _rem, _rem)])

    @pl.when(sid == NS - 1)
    def _():
        tail0 = NS * SUB_ROWS
        pltpu.sync_copy(arows.at[pl.ds(0, TAIL_ROWS)],
                        hsum_sh.at[pl.ds(tail0, TAIL_ROWS)])
        pltpu.sync_copy(degrow.at[pl.ds(0, TAIL_ROWS)],
                        deg_sh.at[pl.ds(tail0, TAIL_ROWS)])

    def _fill_deg(r, _):
        degrow[r, pl.ds(0, LANES)] = one0
        return 0

    lax.fori_loop(0, CHUNK, _fill_deg, 0)

    plsc.subcore_barrier()

    def do_chunk(chunk):
        base = chunk * CHUNK
        pltpu.sync_copy(src_hbm.at[pl.ds(base, CHUNK)], src_v)
        pltpu.sync_copy(tgt_hbm.at[pl.ds(base, CHUNK)], tgt_v)
        g1 = pltpu.async_copy(a_hbm.at[src_v], arows, sem1)
        g2 = pltpu.async_copy(b_hbm.at[tgt_v], brows, sem2)
        pltpu.sync_copy(c_hbm.at[pl.ds(base, CHUNK)], crows)
        g1.wait()
        g2.wait()

        def _row(r, _):
            for cc in range(HIDDEN // LANES):
                s = pl.ds(cc * LANES, LANES)
                arows[r, s] = jnp.maximum(arows[r, s] + brows[r, s] + crows[r, s],
                                          jnp.float32(0.0))
            return 0

        lax.fori_loop(0, CHUNK, _row, 0)

        pltpu.sync_copy(arows, hsum_sh.at[tgt_v], add=True)
        pltpu.sync_copy(degrow, deg_sh.at[tgt_v], add=True)

    def _chunk_loop(i, _):
        do_chunk(wid * BASE_CHUNKS + i)
        return 0

    lax.fori_loop(0, BASE_CHUNKS, _chunk_loop, 0)

    @pl.when(wid < EXTRA)
    def _():
        do_chunk(NW * BASE_CHUNKS + wid)

    plsc.subcore_barrier()

    pltpu.sync_copy(hsum_sh.at[pl.ds(row0, SUB_ROWS)],
                    hsum_out.at[cid, pl.ds(row0, SUB_ROWS)])
    pltpu.sync_copy(deg_sh.at[pl.ds(row0, SUB_ROWS)],
                    deg_out.at[cid, pl.ds(row0, SUB_ROWS)])

    @pl.when(sid == NS - 1)
    def _():
        tail0 = NS * SUB_ROWS
        pltpu.sync_copy(hsum_sh.at[pl.ds(tail0, TAIL_ROWS)],
                        hsum_out.at[cid, pl.ds(tail0, TAIL_ROWS)])
        pltpu.sync_copy(deg_sh.at[pl.ds(tail0, TAIL_ROWS)],
                        deg_out.at[cid, pl.ds(tail0, TAIL_ROWS)])


UPD_GRID = 10
U_BLK = N_NODES // UPD_GRID


def _update_body(nf_ref, p_ref, d_ref, w2t_ref, b2_ref, u1n_ref, u1h_ref,
                 bu1_ref, u2t_ref, bu2_ref, o_ref):
    hsum = p_ref[0] + p_ref[1]
    deg = d_ref[0, :, 0:1] + d_ref[1, :, 0:1]
    agg = (jnp.dot(hsum, w2t_ref[...], preferred_element_type=jnp.float32)
           + deg * b2_ref[...])
    nf = nf_ref[...]
    u = jnp.maximum(
        jnp.dot(nf, u1n_ref[...], preferred_element_type=jnp.float32)
        + jnp.dot(agg, u1h_ref[...], preferred_element_type=jnp.float32)
        + bu1_ref[...], jnp.float32(0.0))
    o_ref[...] = (nf + jnp.dot(u, u2t_ref[...],
                               preferred_element_type=jnp.float32)
                  + bu2_ref[...])


_update = pl.pallas_call(
    _update_body,
    grid=(UPD_GRID,),
    in_specs=[
        pl.BlockSpec((U_BLK, NODE_DIM), lambda i: (i, 0)),
        pl.BlockSpec((NC, U_BLK, HIDDEN), lambda i: (0, i, 0)),
        pl.BlockSpec((NC, U_BLK, DEG_W), lambda i: (0, i, 0)),
        pl.BlockSpec((HIDDEN, HIDDEN), lambda i: (0, 0)),
        pl.BlockSpec((1, HIDDEN), lambda i: (0, 0)),
        pl.BlockSpec((NODE_DIM, HIDDEN), lambda i: (0, 0)),
        pl.BlockSpec((HIDDEN, HIDDEN), lambda i: (0, 0)),
        pl.BlockSpec((1, HIDDEN), lambda i: (0, 0)),
        pl.BlockSpec((HIDDEN, NODE_DIM), lambda i: (0, 0)),
        pl.BlockSpec((1, NODE_DIM), lambda i: (0, 0)),
    ],
    out_specs=pl.BlockSpec((U_BLK, NODE_DIM), lambda i: (i, 0)),
    out_shape=jax.ShapeDtypeStruct((N_NODES, NODE_DIM), jnp.float32),
)


def kernel(node_feats, edge_feats, edge_index, W1, b1, W2, b2, U1, bu1, U2, bu2):
    ei = edge_index.astype(jnp.int32)
    src = ei[0]
    tgt = ei[1]
    w1s = W1[:, :NODE_DIM].T
    w1t = W1[:, NODE_DIM:2 * NODE_DIM].T
    w1e = W1[:, 2 * NODE_DIM:].T
    a_tab, b_tab, c_rows = _prep(node_feats, edge_feats, w1s, w1t, w1e,
                                 b1.reshape(1, HIDDEN))
    hsum, deg = _sc_aggregate(a_tab, b_tab, c_rows, src, tgt)
    return _update(node_feats, hsum, deg, W2.T, b2.reshape(1, HIDDEN),
                   U1[:, :NODE_DIM].T, U1[:, NODE_DIM:].T,
                   bu1.reshape(1, HIDDEN), U2.T, bu2.reshape(1, NODE_DIM))


# speedup vs baseline: 3.5053x; 1.0516x over previous
import functools

import jax
import jax.numpy as jnp
from jax import lax
from jax.experimental import pallas as pl
from jax.experimental.pallas import tpu as pltpu
from jax.experimental.pallas import tpu_sc as plsc

N_NODES = 10000
NODE_DIM = 128
HIDDEN = 128
EDGE_DIM = 16
N_EDGES = 320000

NC, NS = 2, 16
NW = NC * NS
CHUNK = 40
W_CHUNKS = N_EDGES // (NW * CHUNK)
NITER = W_CHUNKS // 2
SUB_ROWS = 624
TAIL_ROWS = N_NODES - NS * SUB_ROWS
DEG_W = 16
LANES = 16


PREP_GRID = 125
E_BLK = N_EDGES // PREP_GRID
N_BLK = N_NODES // PREP_GRID


def _prep_body(nf_ref, ef_ref, w1s_ref, w1t_ref, w1e_ref, b1_ref,
               a_ref, b_ref, c_ref):
    nf = nf_ref[...]
    a_ref[...] = jnp.dot(nf, w1s_ref[...], preferred_element_type=jnp.float32)
    b_ref[...] = jnp.dot(nf, w1t_ref[...], preferred_element_type=jnp.float32)
    c_ref[...] = (jnp.dot(ef_ref[...], w1e_ref[...],
                          preferred_element_type=jnp.float32) + b1_ref[...])


_prep = pl.pallas_call(
    _prep_body,
    grid=(PREP_GRID,),
    in_specs=[
        pl.BlockSpec((N_BLK, NODE_DIM), lambda i: (i, 0)),
        pl.BlockSpec((E_BLK, EDGE_DIM), lambda i: (i, 0)),
        pl.BlockSpec((NODE_DIM, HIDDEN), lambda i: (0, 0)),
        pl.BlockSpec((NODE_DIM, HIDDEN), lambda i: (0, 0)),
        pl.BlockSpec((EDGE_DIM, HIDDEN), lambda i: (0, 0)),
        pl.BlockSpec((1, HIDDEN), lambda i: (0, 0)),
    ],
    out_specs=[
        pl.BlockSpec((N_BLK, HIDDEN), lambda i: (i, 0)),
        pl.BlockSpec((N_BLK, HIDDEN), lambda i: (i, 0)),
        pl.BlockSpec((E_BLK, HIDDEN), lambda i: (i, 0)),
    ],
    out_shape=[
        jax.ShapeDtypeStruct((N_NODES, HIDDEN), jnp.float32),
        jax.ShapeDtypeStruct((N_NODES, HIDDEN), jnp.float32),
        jax.ShapeDtypeStruct((N_EDGES, HIDDEN), jnp.float32),
    ],
)


_sc_mesh = plsc.VectorSubcoreMesh(core_axis_name="c", subcore_axis_name="s")


@functools.partial(
    pl.kernel,
    mesh=_sc_mesh,
    compiler_params=pltpu.CompilerParams(use_tc_tiling_on_sc=False),
    out_type=[
        pltpu.HBM((NC, N_NODES, HIDDEN), jnp.float32),
        pltpu.HBM((NC, N_NODES, DEG_W), jnp.float32),
    ],
    scratch_types=[
        pltpu.VMEM((CHUNK,), jnp.int32),
        pltpu.VMEM((CHUNK,), jnp.int32),
        pltpu.VMEM((CHUNK,), jnp.int32),
        pltpu.VMEM((CHUNK,), jnp.int32),
        pltpu.VMEM((CHUNK, HIDDEN), jnp.float32),
        pltpu.VMEM((CHUNK, HIDDEN), jnp.float32),
        pltpu.VMEM((CHUNK, HIDDEN), jnp.float32),
        pltpu.VMEM((CHUNK, HIDDEN), jnp.float32),
        pltpu.VMEM((CHUNK, HIDDEN), jnp.float32),
        pltpu.VMEM((CHUNK, HIDDEN), jnp.float32),
        pltpu.VMEM((CHUNK, DEG_W), jnp.float32),
        pltpu.VMEM_SHARED((N_NODES, HIDDEN), jnp.float32),
        pltpu.VMEM_SHARED((N_NODES, DEG_W), jnp.float32),
        pltpu.SemaphoreType.DMA,
        pltpu.SemaphoreType.DMA,
        pltpu.SemaphoreType.DMA,
        pltpu.SemaphoreType.DMA,
    ],
)
def _sc_aggregate(a_hbm, b_hbm, c_hbm, src_hbm, tgt_hbm,
                  hsum_out, deg_out,
                  src0, tgt0, src1, tgt1,
                  ar0, br0, cr0, ar1, br1, cr1,
                  degrow, hsum_sh, deg_sh,
                  gsem0, gsem1, isem0, isem1):
    cid = lax.axis_index("c")
    sid = lax.axis_index("s")
    wid = sid * NC + cid
    sets = ((src0, tgt0, ar0, br0, cr0, gsem0, isem0),
            (src1, tgt1, ar1, br1, cr1, gsem1, isem1))

    lanes = lax.iota(jnp.int32, LANES)
    zero16 = jnp.zeros((LANES,), jnp.float32)
    one0 = jnp.where(lanes == 0, jnp.float32(1.0), jnp.float32(0.0))

    def _fill_zeros(r, _):
        for cc in range(HIDDEN // LANES):
            ar0[r, pl.ds(cc * LANES, LANES)] = zero16
        degrow[r, pl.ds(0, LANES)] = zero16
        return 0

    lax.fori_loop(0, CHUNK, _fill_zeros, 0)

    row0 = sid * SUB_ROWS
    for z in range(SUB_ROWS // CHUNK):
        pltpu.sync_copy(ar0, hsum_sh.at[pl.ds(row0 + z * CHUNK, CHUNK)])
        pltpu.sync_copy(degrow, deg_sh.at[pl.ds(row0 + z * CHUNK, CHUNK)])
    _rem = SUB_ROWS - (SUB_ROWS // CHUNK) * CHUNK
    pltpu.sync_copy(ar0.at[pl.ds(0, _rem)],
                    hsum_sh.at[pl.ds(row0 + SUB_ROWS - _rem, _rem)])
    pltpu.sync_copy(degrow.at[pl.ds(0, _rem)],
                    deg_sh.at[pl.ds(row0 + SUB_ROWS - _rem, _rem)])

    @pl.when(sid == NS - 1)
    def _():
        tail0 = NS * SUB_ROWS
        pltpu.sync_copy(ar0.at[pl.ds(0, TAIL_ROWS)],
                        hsum_sh.at[pl.ds(tail0, TAIL_ROWS)])
        pltpu.sync_copy(degrow.at[pl.ds(0, TAIL_ROWS)],
                        deg_sh.at[pl.ds(tail0, TAIL_ROWS)])

    def _fill_deg(r, _):
        degrow[r, pl.ds(0, LANES)] = one0
        return 0

    lax.fori_loop(0, CHUNK, _fill_deg, 0)

    plsc.subcore_barrier()

    def cbase(j):
        return (wid * W_CHUNKS + j) * CHUNK

    pltpu.sync_copy(src_hbm.at[pl.ds(cbase(0), CHUNK)], src0)
    pltpu.sync_copy(tgt_hbm.at[pl.ds(cbase(0), CHUNK)], tgt0)
    pltpu.async_copy(src_hbm.at[pl.ds(cbase(1), CHUNK)], src1, isem1)
    pltpu.async_copy(tgt_hbm.at[pl.ds(cbase(1), CHUNK)], tgt1, isem1)
    pltpu.async_copy(a_hbm.at[src0], ar0, gsem0)
    pltpu.async_copy(b_hbm.at[tgt0], br0, gsem0)
    pltpu.async_copy(c_hbm.at[pl.ds(cbase(0), CHUNK)], cr0, gsem0)

    def _iter(i, _):
        for k in (0, 1):
            srcv, tgtv, ar, br, cr, gsem, isem = sets[k]
            osrc, otgt, oar, obr, ocr, ogsem, oisem = sets[1 - k]
            j = i * 2 + k
            pltpu.make_async_copy(a_hbm.at[srcv], ar, gsem).wait()
            pltpu.make_async_copy(b_hbm.at[tgtv], br, gsem).wait()
            pltpu.make_async_copy(c_hbm.at[pl.ds(cbase(j), CHUNK)], cr,
                                  gsem).wait()

            def _row(r, _):
                for cc in range(HIDDEN // LANES):
                    sl = pl.ds(cc * LANES, LANES)
                    ar[r, sl] = jnp.maximum(ar[r, sl] + br[r, sl] + cr[r, sl],
                                            jnp.float32(0.0))
                return 0

            lax.fori_loop(0, CHUNK, _row, 0)

            pltpu.sync_copy(ar, hsum_sh.at[tgtv], add=True)
            pltpu.sync_copy(degrow, deg_sh.at[tgtv], add=True)

            @pl.when(i < NITER - 1)
            def _():
                pltpu.async_copy(src_hbm.at[pl.ds(cbase(j + 2), CHUNK)],
                                 srcv, isem)
                pltpu.async_copy(tgt_hbm.at[pl.ds(cbase(j + 2), CHUNK)],
                                 tgtv, isem)

            def _fire_next():
                pltpu.make_async_copy(
                    src_hbm.at[pl.ds(cbase(j + 1), CHUNK)], osrc, oisem).wait()
                pltpu.make_async_copy(
                    tgt_hbm.at[pl.ds(cbase(j + 1), CHUNK)], otgt, oisem).wait()
                pltpu.async_copy(a_hbm.at[osrc], oar, ogsem)
                pltpu.async_copy(b_hbm.at[otgt], obr, ogsem)
                pltpu.async_copy(c_hbm.at[pl.ds(cbase(j + 1), CHUNK)],
                                 ocr, ogsem)

            if k == 0:
                _fire_next()
            else:
                pl.when(i < NITER - 1)(_fire_next)
        return 0

    lax.fori_loop(0, NITER, _iter, 0)

    plsc.subcore_barrier()

    pltpu.sync_copy(hsum_sh.at[pl.ds(row0, SUB_ROWS)],
                    hsum_out.at[cid, pl.ds(row0, SUB_ROWS)])
    pltpu.sync_copy(deg_sh.at[pl.ds(row0, SUB_ROWS)],
                    deg_out.at[cid, pl.ds(row0, SUB_ROWS)])

    @pl.when(sid == NS - 1)
    def _():
        tail0 = NS * SUB_ROWS
        pltpu.sync_copy(hsum_sh.at[pl.ds(tail0, TAIL_ROWS)],
                        hsum_out.at[cid, pl.ds(tail0, TAIL_ROWS)])
        pltpu.sync_copy(deg_sh.at[pl.ds(tail0, TAIL_ROWS)],
                        deg_out.at[cid, pl.ds(tail0, TAIL_ROWS)])


UPD_GRID = 10
U_BLK = N_NODES // UPD_GRID


def _update_body(nf_ref, p_ref, d_ref, w2t_ref, b2_ref, u1n_ref, u1h_ref,
                 bu1_ref, u2t_ref, bu2_ref, o_ref):
    hsum = p_ref[0] + p_ref[1]
    deg = d_ref[0, :, 0:1] + d_ref[1, :, 0:1]
    agg = (jnp.dot(hsum, w2t_ref[...], preferred_element_type=jnp.float32)
           + deg * b2_ref[...])
    nf = nf_ref[...]
    u = jnp.maximum(
        jnp.dot(nf, u1n_ref[...], preferred_element_type=jnp.float32)
        + jnp.dot(agg, u1h_ref[...], preferred_element_type=jnp.float32)
        + bu1_ref[...], jnp.float32(0.0))
    o_ref[...] = (nf + jnp.dot(u, u2t_ref[...],
                               preferred_element_type=jnp.float32)
                  + bu2_ref[...])


_update = pl.pallas_call(
    _update_body,
    grid=(UPD_GRID,),
    in_specs=[
        pl.BlockSpec((U_BLK, NODE_DIM), lambda i: (i, 0)),
        pl.BlockSpec((NC, U_BLK, HIDDEN), lambda i: (0, i, 0)),
        pl.BlockSpec((NC, U_BLK, DEG_W), lambda i: (0, i, 0)),
        pl.BlockSpec((HIDDEN, HIDDEN), lambda i: (0, 0)),
        pl.BlockSpec((1, HIDDEN), lambda i: (0, 0)),
        pl.BlockSpec((NODE_DIM, HIDDEN), lambda i: (0, 0)),
        pl.BlockSpec((HIDDEN, HIDDEN), lambda i: (0, 0)),
        pl.BlockSpec((1, HIDDEN), lambda i: (0, 0)),
        pl.BlockSpec((HIDDEN, NODE_DIM), lambda i: (0, 0)),
        pl.BlockSpec((1, NODE_DIM), lambda i: (0, 0)),
    ],
    out_specs=pl.BlockSpec((U_BLK, NODE_DIM), lambda i: (i, 0)),
    out_shape=jax.ShapeDtypeStruct((N_NODES, NODE_DIM), jnp.float32),
)


def kernel(node_feats, edge_feats, edge_index, W1, b1, W2, b2, U1, bu1, U2, bu2):
    ei = edge_index.astype(jnp.int32)
    src = ei[0]
    tgt = ei[1]
    w1s = W1[:, :NODE_DIM].T
    w1t = W1[:, NODE_DIM:2 * NODE_DIM].T
    w1e = W1[:, 2 * NODE_DIM:].T
    a_tab, b_tab, c_rows = _prep(node_feats, edge_feats, w1s, w1t, w1e,
                                 b1.reshape(1, HIDDEN))
    hsum, deg = _sc_aggregate(a_tab, b_tab, c_rows, src, tgt)
    return _update(node_feats, hsum, deg, W2.T, b2.reshape(1, HIDDEN),
                   U1[:, :NODE_DIM].T, U1[:, NODE_DIM:].T,
                   bu1.reshape(1, HIDDEN), U2.T, bu2.reshape(1, NODE_DIM))


# speedup vs baseline: 3.8578x; 1.1006x over previous
import functools

import jax
import jax.numpy as jnp
from jax import lax
from jax.experimental import pallas as pl
from jax.experimental.pallas import tpu as pltpu
from jax.experimental.pallas import tpu_sc as plsc

N_NODES = 10000
NODE_DIM = 128
HIDDEN = 128
EDGE_DIM = 16
N_EDGES = 320000

NC, NS = 2, 16
NW = NC * NS
CHUNK = 40
W_CHUNKS = N_EDGES // (NW * CHUNK)
NITER = W_CHUNKS // 2
SUB_ROWS = 624
TAIL_ROWS = N_NODES - NS * SUB_ROWS
DEG_W = 16
LANES = 16


PREP_GRID = 125
E_BLK = N_EDGES // PREP_GRID
N_BLK = N_NODES // PREP_GRID


def _prep_body(nf_ref, ef_ref, w1s_ref, w1t_ref, w1e_ref, b1_ref,
               a_ref, b_ref, c_ref):
    nf = nf_ref[...]
    a_ref[...] = jnp.dot(nf, w1s_ref[...], preferred_element_type=jnp.float32)
    b_ref[...] = jnp.dot(nf, w1t_ref[...], preferred_element_type=jnp.float32)
    c_ref[...] = (jnp.dot(ef_ref[...], w1e_ref[...],
                          preferred_element_type=jnp.float32) + b1_ref[...])


_prep = pl.pallas_call(
    _prep_body,
    grid=(PREP_GRID,),
    in_specs=[
        pl.BlockSpec((N_BLK, NODE_DIM), lambda i: (i, 0)),
        pl.BlockSpec((E_BLK, EDGE_DIM), lambda i: (i, 0)),
        pl.BlockSpec((NODE_DIM, HIDDEN), lambda i: (0, 0)),
        pl.BlockSpec((NODE_DIM, HIDDEN), lambda i: (0, 0)),
        pl.BlockSpec((EDGE_DIM, HIDDEN), lambda i: (0, 0)),
        pl.BlockSpec((1, HIDDEN), lambda i: (0, 0)),
    ],
    out_specs=[
        pl.BlockSpec((N_BLK, HIDDEN), lambda i: (i, 0)),
        pl.BlockSpec((N_BLK, HIDDEN), lambda i: (i, 0)),
        pl.BlockSpec((E_BLK, HIDDEN), lambda i: (i, 0)),
    ],
    out_shape=[
        jax.ShapeDtypeStruct((N_NODES, HIDDEN), jnp.float32),
        jax.ShapeDtypeStruct((N_NODES, HIDDEN), jnp.float32),
        jax.ShapeDtypeStruct((N_EDGES, HIDDEN), jnp.float32),
    ],
)


_sc_mesh = plsc.VectorSubcoreMesh(core_axis_name="c", subcore_axis_name="s")


@functools.partial(
    pl.kernel,
    mesh=_sc_mesh,
    compiler_params=pltpu.CompilerParams(use_tc_tiling_on_sc=False),
    out_type=[
        pltpu.HBM((NC, N_NODES, HIDDEN), jnp.float32),
        pltpu.HBM((NC, N_NODES, DEG_W), jnp.float32),
    ],
    scratch_types=[
        pltpu.VMEM((CHUNK,), jnp.int32),
        pltpu.VMEM((CHUNK,), jnp.int32),
        pltpu.VMEM((CHUNK,), jnp.int32),
        pltpu.VMEM((CHUNK,), jnp.int32),
        pltpu.VMEM((CHUNK,), jnp.int32),
        pltpu.VMEM((CHUNK,), jnp.int32),
        pltpu.VMEM((CHUNK, HIDDEN), jnp.float32),
        pltpu.VMEM((CHUNK, HIDDEN), jnp.float32),
        pltpu.VMEM((CHUNK, HIDDEN), jnp.float32),
        pltpu.VMEM((CHUNK, HIDDEN), jnp.float32),
        pltpu.VMEM((CHUNK, HIDDEN), jnp.float32),
        pltpu.VMEM((CHUNK, HIDDEN), jnp.float32),
        pltpu.VMEM((CHUNK, DEG_W), jnp.float32),
        pltpu.VMEM_SHARED((N_NODES, HIDDEN), jnp.float32),
        pltpu.VMEM_SHARED((N_NODES, DEG_W), jnp.float32),
        pltpu.SemaphoreType.DMA,
        pltpu.SemaphoreType.DMA,
        pltpu.SemaphoreType.DMA,
        pltpu.SemaphoreType.DMA,
        pltpu.SemaphoreType.DMA,
        pltpu.SemaphoreType.DMA,
    ],
)
def _sc_aggregate(a_hbm, b_hbm, c_hbm, src_hbm, tgt_hbm,
                  hsum_out, deg_out,
                  src0, tgt0, src1, tgt1, stgt0, stgt1,
                  ar0, br0, cr0, ar1, br1, cr1,
                  degrow, hsum_sh, deg_sh,
                  gsem0, gsem1, isem0, isem1, ssem0, ssem1):
    cid = lax.axis_index("c")
    sid = lax.axis_index("s")
    wid = sid * NC + cid
    sets = ((src0, tgt0, stgt0, ar0, br0, cr0, gsem0, isem0, ssem0),
            (src1, tgt1, stgt1, ar1, br1, cr1, gsem1, isem1, ssem1))

    lanes = lax.iota(jnp.int32, LANES)
    zero16 = jnp.zeros((LANES,), jnp.float32)
    one0 = jnp.where(lanes == 0, jnp.float32(1.0), jnp.float32(0.0))

    def _fill_zeros(r, _):
        for cc in range(HIDDEN // LANES):
            ar0[r, pl.ds(cc * LANES, LANES)] = zero16
        degrow[r, pl.ds(0, LANES)] = zero16
        return 0

    lax.fori_loop(0, CHUNK, _fill_zeros, 0)

    row0 = sid * SUB_ROWS
    for z in range(SUB_ROWS // CHUNK):
        pltpu.sync_copy(ar0, hsum_sh.at[pl.ds(row0 + z * CHUNK, CHUNK)])
        pltpu.sync_copy(degrow, deg_sh.at[pl.ds(row0 + z * CHUNK, CHUNK)])
    _rem = SUB_ROWS - (SUB_ROWS // CHUNK) * CHUNK
    pltpu.sync_copy(ar0.at[pl.ds(0, _rem)],
                    hsum_sh.at[pl.ds(row0 + SUB_ROWS - _rem, _rem)])
    pltpu.sync_copy(degrow.at[pl.ds(0, _rem)],
                    deg_sh.at[pl.ds(row0 + SUB_ROWS - _rem, _rem)])

    @pl.when(sid == NS - 1)
    def _():
        tail0 = NS * SUB_ROWS
        pltpu.sync_copy(ar0.at[pl.ds(0, TAIL_ROWS)],
                        hsum_sh.at[pl.ds(tail0, TAIL_ROWS)])
        pltpu.sync_copy(degrow.at[pl.ds(0, TAIL_ROWS)],
                        deg_sh.at[pl.ds(tail0, TAIL_ROWS)])

    def _fill_deg(r, _):
        degrow[r, pl.ds(0, LANES)] = one0
        return 0

    lax.fori_loop(0, CHUNK, _fill_deg, 0)

    plsc.subcore_barrier()

    def cbase(j):
        return (wid * W_CHUNKS + j) * CHUNK

    pltpu.sync_copy(src_hbm.at[pl.ds(cbase(0), CHUNK)], src0)
    pltpu.sync_copy(tgt_hbm.at[pl.ds(cbase(0), CHUNK)], tgt0)
    pltpu.async_copy(src_hbm.at[pl.ds(cbase(1), CHUNK)], src1, isem1)
    pltpu.async_copy(tgt_hbm.at[pl.ds(cbase(1), CHUNK)], tgt1, isem1)
    pltpu.async_copy(a_hbm.at[src0], ar0, gsem0)
    pltpu.async_copy(b_hbm.at[tgt0], br0, gsem0)
    pltpu.async_copy(c_hbm.at[pl.ds(cbase(0), CHUNK)], cr0, gsem0)

    def _iter(i, _):
        for k in (0, 1):
            srcv, tgtv, stgt, ar, br, cr, gsem, isem, ssem = sets[k]
            (osrc, otgt, ostgt, oar, obr, ocr,
             ogsem, oisem, ossem) = sets[1 - k]
            j = i * 2 + k
            pltpu.make_async_copy(a_hbm.at[srcv], ar, gsem).wait()
            pltpu.make_async_copy(b_hbm.at[tgtv], br, gsem).wait()
            pltpu.make_async_copy(c_hbm.at[pl.ds(cbase(j), CHUNK)], cr,
                                  gsem).wait()

            @plsc.parallel_loop(0, CHUNK, unroll=4)
            def _row(r):
                for cc in range(HIDDEN // LANES):
                    sl = pl.ds(cc * LANES, LANES)
                    ar[r, sl] = jnp.maximum(ar[r, sl] + br[r, sl] + cr[r, sl],
                                            jnp.float32(0.0))

            stgt[pl.ds(0, LANES)] = tgtv[pl.ds(0, LANES)]
            stgt[pl.ds(LANES, LANES)] = tgtv[pl.ds(LANES, LANES)]
            stgt[pl.ds(CHUNK - LANES, LANES)] = tgtv[pl.ds(CHUNK - LANES,
                                                           LANES)]
            pltpu.async_copy(ar, hsum_sh.at[stgt], ssem, add=True)
            pltpu.async_copy(degrow, deg_sh.at[stgt], ssem, add=True)

            @pl.when(i < NITER - 1)
            def _():
                pltpu.async_copy(src_hbm.at[pl.ds(cbase(j + 2), CHUNK)],
                                 srcv, isem)
                pltpu.async_copy(tgt_hbm.at[pl.ds(cbase(j + 2), CHUNK)],
                                 tgtv, isem)

            def _drain_other_scatter():
                pltpu.make_async_copy(oar, hsum_sh.at[ostgt], ossem).wait()
                pltpu.make_async_copy(degrow, deg_sh.at[ostgt], ossem).wait()

            def _fire_next():
                pltpu.make_async_copy(
                    src_hbm.at[pl.ds(cbase(j + 1), CHUNK)], osrc, oisem).wait()
                pltpu.make_async_copy(
                    tgt_hbm.at[pl.ds(cbase(j + 1), CHUNK)], otgt, oisem).wait()
                pltpu.async_copy(a_hbm.at[osrc], oar, ogsem)
                pltpu.async_copy(b_hbm.at[otgt], obr, ogsem)
                pltpu.async_copy(c_hbm.at[pl.ds(cbase(j + 1), CHUNK)],
                                 ocr, ogsem)

            if k == 0:
                pl.when(i > 0)(_drain_other_scatter)
                _fire_next()
            else:
                def _drain_and_fire():
                    _drain_other_scatter()
                    _fire_next()
                pl.when(i < NITER - 1)(_drain_and_fire)
        return 0

    lax.fori_loop(0, NITER, _iter, 0)

    pltpu.make_async_copy(ar0, hsum_sh.at[stgt0], ssem0).wait()
    pltpu.make_async_copy(degrow, deg_sh.at[stgt0], ssem0).wait()
    pltpu.make_async_copy(ar1, hsum_sh.at[stgt1], ssem1).wait()
    pltpu.make_async_copy(degrow, deg_sh.at[stgt1], ssem1).wait()

    plsc.subcore_barrier()

    pltpu.sync_copy(hsum_sh.at[pl.ds(row0, SUB_ROWS)],
                    hsum_out.at[cid, pl.ds(row0, SUB_ROWS)])
    pltpu.sync_copy(deg_sh.at[pl.ds(row0, SUB_ROWS)],
                    deg_out.at[cid, pl.ds(row0, SUB_ROWS)])

    @pl.when(sid == NS - 1)
    def _():
        tail0 = NS * SUB_ROWS
        pltpu.sync_copy(hsum_sh.at[pl.ds(tail0, TAIL_ROWS)],
                        hsum_out.at[cid, pl.ds(tail0, TAIL_ROWS)])
        pltpu.sync_copy(deg_sh.at[pl.ds(tail0, TAIL_ROWS)],
                        deg_out.at[cid, pl.ds(tail0, TAIL_ROWS)])


UPD_GRID = 10
U_BLK = N_NODES // UPD_GRID


def _update_body(nf_ref, p_ref, d_ref, w2t_ref, b2_ref, u1n_ref, u1h_ref,
                 bu1_ref, u2t_ref, bu2_ref, o_ref):
    hsum = p_ref[0] + p_ref[1]
    deg = d_ref[0, :, 0:1] + d_ref[1, :, 0:1]
    agg = (jnp.dot(hsum, w2t_ref[...], preferred_element_type=jnp.float32)
           + deg * b2_ref[...])
    nf = nf_ref[...]
    u = jnp.maximum(
        jnp.dot(nf, u1n_ref[...], preferred_element_type=jnp.float32)
        + jnp.dot(agg, u1h_ref[...], preferred_element_type=jnp.float32)
        + bu1_ref[...], jnp.float32(0.0))
    o_ref[...] = (nf + jnp.dot(u, u2t_ref[...],
                               preferred_element_type=jnp.float32)
                  + bu2_ref[...])


_update = pl.pallas_call(
    _update_body,
    grid=(UPD_GRID,),
    in_specs=[
        pl.BlockSpec((U_BLK, NODE_DIM), lambda i: (i, 0)),
        pl.BlockSpec((NC, U_BLK, HIDDEN), lambda i: (0, i, 0)),
        pl.BlockSpec((NC, U_BLK, DEG_W), lambda i: (0, i, 0)),
        pl.BlockSpec((HIDDEN, HIDDEN), lambda i: (0, 0)),
        pl.BlockSpec((1, HIDDEN), lambda i: (0, 0)),
        pl.BlockSpec((NODE_DIM, HIDDEN), lambda i: (0, 0)),
        pl.BlockSpec((HIDDEN, HIDDEN), lambda i: (0, 0)),
        pl.BlockSpec((1, HIDDEN), lambda i: (0, 0)),
        pl.BlockSpec((HIDDEN, NODE_DIM), lambda i: (0, 0)),
        pl.BlockSpec((1, NODE_DIM), lambda i: (0, 0)),
    ],
    out_specs=pl.BlockSpec((U_BLK, NODE_DIM), lambda i: (i, 0)),
    out_shape=jax.ShapeDtypeStruct((N_NODES, NODE_DIM), jnp.float32),
)


def kernel(node_feats, edge_feats, edge_index, W1, b1, W2, b2, U1, bu1, U2, bu2):
    ei = edge_index.astype(jnp.int32)
    src = ei[0]
    tgt = ei[1]
    w1s = W1[:, :NODE_DIM].T
    w1t = W1[:, NODE_DIM:2 * NODE_DIM].T
    w1e = W1[:, 2 * NODE_DIM:].T
    a_tab, b_tab, c_rows = _prep(node_feats, edge_feats, w1s, w1t, w1e,
                                 b1.reshape(1, HIDDEN))
    hsum, deg = _sc_aggregate(a_tab, b_tab, c_rows, src, tgt)
    return _update(node_feats, hsum, deg, W2.T, b2.reshape(1, HIDDEN),
                   U1[:, :NODE_DIM].T, U1[:, NODE_DIM:].T,
                   bu1.reshape(1, HIDDEN), U2.T, bu2.reshape(1, NODE_DIM))


# speedup vs baseline: 4.6626x; 1.2086x over previous
import functools

import jax
import jax.numpy as jnp
from jax import lax
from jax.experimental import pallas as pl
from jax.experimental.pallas import tpu as pltpu
from jax.experimental.pallas import tpu_sc as plsc

N_NODES = 10000
NODE_DIM = 128
HIDDEN = 128
EDGE_DIM = 16
N_EDGES = 320000

NC, NS = 2, 16
NW = NC * NS
CHUNK = 40
W_CHUNKS = N_EDGES // (NW * CHUNK)
NITER = W_CHUNKS // 2
SUB_ROWS = 624
TAIL_ROWS = N_NODES - NS * SUB_ROWS
DEG_W = 16
LANES = 16


PREP_GRID = 125
E_BLK = N_EDGES // PREP_GRID
N_BLK = N_NODES // PREP_GRID


def _prep_body(nf_ref, ef_ref, w1s_ref, w1t_ref, w1e_ref, b1_ref,
               a_ref, b_ref, c_ref):
    nf = nf_ref[...]
    a_ref[...] = jnp.dot(nf, w1s_ref[...], preferred_element_type=jnp.float32)
    b_ref[...] = jnp.dot(nf, w1t_ref[...], preferred_element_type=jnp.float32)
    c_ref[...] = (jnp.dot(ef_ref[...], w1e_ref[...],
                          preferred_element_type=jnp.float32) + b1_ref[...])


_prep = pl.pallas_call(
    _prep_body,
    grid=(PREP_GRID,),
    in_specs=[
        pl.BlockSpec((N_BLK, NODE_DIM), lambda i: (i, 0)),
        pl.BlockSpec((E_BLK, EDGE_DIM), lambda i: (i, 0)),
        pl.BlockSpec((NODE_DIM, HIDDEN), lambda i: (0, 0)),
        pl.BlockSpec((NODE_DIM, HIDDEN), lambda i: (0, 0)),
        pl.BlockSpec((EDGE_DIM, HIDDEN), lambda i: (0, 0)),
        pl.BlockSpec((1, HIDDEN), lambda i: (0, 0)),
    ],
    out_specs=[
        pl.BlockSpec((N_BLK, HIDDEN), lambda i: (i, 0)),
        pl.BlockSpec((N_BLK, HIDDEN), lambda i: (i, 0)),
        pl.BlockSpec((E_BLK, HIDDEN), lambda i: (i, 0)),
    ],
    out_shape=[
        jax.ShapeDtypeStruct((N_NODES, HIDDEN), jnp.float32),
        jax.ShapeDtypeStruct((N_NODES, HIDDEN), jnp.float32),
        jax.ShapeDtypeStruct((N_EDGES, HIDDEN), jnp.float32),
    ],
)


_sc_mesh = plsc.VectorSubcoreMesh(core_axis_name="c", subcore_axis_name="s")


@functools.partial(
    pl.kernel,
    mesh=_sc_mesh,
    compiler_params=pltpu.CompilerParams(use_tc_tiling_on_sc=False),
    out_type=[
        pltpu.HBM((NC, N_NODES, HIDDEN), jnp.float32),
        pltpu.HBM((NC, N_NODES, DEG_W), jnp.float32),
    ],
    scratch_types=[
        pltpu.VMEM((CHUNK,), jnp.int32),
        pltpu.VMEM((CHUNK,), jnp.int32),
        pltpu.VMEM((CHUNK,), jnp.int32),
        pltpu.VMEM((CHUNK,), jnp.int32),
        pltpu.VMEM((CHUNK,), jnp.int32),
        pltpu.VMEM((CHUNK,), jnp.int32),
        pltpu.VMEM((CHUNK, HIDDEN), jnp.float32),
        pltpu.VMEM((CHUNK, HIDDEN), jnp.float32),
        pltpu.VMEM((CHUNK, HIDDEN), jnp.float32),
        pltpu.VMEM((CHUNK, HIDDEN), jnp.float32),
        pltpu.VMEM((CHUNK, HIDDEN), jnp.float32),
        pltpu.VMEM((CHUNK, HIDDEN), jnp.float32),
        pltpu.VMEM((CHUNK, DEG_W), jnp.float32),
        pltpu.VMEM_SHARED((N_NODES, HIDDEN), jnp.float32),
        pltpu.VMEM_SHARED((N_NODES, DEG_W), jnp.float32),
        pltpu.SemaphoreType.DMA,
        pltpu.SemaphoreType.DMA,
        pltpu.SemaphoreType.DMA,
        pltpu.SemaphoreType.DMA,
        pltpu.SemaphoreType.DMA,
        pltpu.SemaphoreType.DMA,
    ],
)
def _sc_aggregate(a_hbm, b_hbm, c_hbm, src_hbm, tgt_hbm,
                  hsum_out, deg_out,
                  src0, tgt0, src1, tgt1, stgt0, stgt1,
                  ar0, br0, cr0, ar1, br1, cr1,
                  degrow, hsum_sh, deg_sh,
                  gsem0, gsem1, isem0, isem1, ssem0, ssem1):
    cid = lax.axis_index("c")
    sid = lax.axis_index("s")
    wid = sid * NC + cid
    sets = ((src0, tgt0, stgt0, ar0, br0, cr0, gsem0, isem0, ssem0),
            (src1, tgt1, stgt1, ar1, br1, cr1, gsem1, isem1, ssem1))

    lanes = lax.iota(jnp.int32, LANES)
    zero16 = jnp.zeros((LANES,), jnp.float32)
    one0 = jnp.where(lanes == 0, jnp.float32(1.0), jnp.float32(0.0))

    def _fill_zeros(r, _):
        for cc in range(HIDDEN // LANES):
            ar0[r, pl.ds(cc * LANES, LANES)] = zero16
        degrow[r, pl.ds(0, LANES)] = zero16
        return 0

    lax.fori_loop(0, CHUNK, _fill_zeros, 0)

    row0 = sid * SUB_ROWS
    for z in range(SUB_ROWS // CHUNK):
        pltpu.sync_copy(ar0, hsum_sh.at[pl.ds(row0 + z * CHUNK, CHUNK)])
        pltpu.sync_copy(degrow, deg_sh.at[pl.ds(row0 + z * CHUNK, CHUNK)])
    _rem = SUB_ROWS - (SUB_ROWS // CHUNK) * CHUNK
    pltpu.sync_copy(ar0.at[pl.ds(0, _rem)],
                    hsum_sh.at[pl.ds(row0 + SUB_ROWS - _rem, _rem)])
    pltpu.sync_copy(degrow.at[pl.ds(0, _rem)],
                    deg_sh.at[pl.ds(row0 + SUB_ROWS - _rem, _rem)])

    @pl.when(sid == NS - 1)
    def _():
        tail0 = NS * SUB_ROWS
        pltpu.sync_copy(ar0.at[pl.ds(0, TAIL_ROWS)],
                        hsum_sh.at[pl.ds(tail0, TAIL_ROWS)])
        pltpu.sync_copy(degrow.at[pl.ds(0, TAIL_ROWS)],
                        deg_sh.at[pl.ds(tail0, TAIL_ROWS)])

    def _fill_deg(r, _):
        degrow[r, pl.ds(0, LANES)] = one0
        return 0

    lax.fori_loop(0, CHUNK, _fill_deg, 0)

    plsc.subcore_barrier()

    def cbase(j):
        return (wid * W_CHUNKS + j) * CHUNK

    pltpu.sync_copy(src_hbm.at[pl.ds(cbase(0), CHUNK)], src0)
    pltpu.sync_copy(tgt_hbm.at[pl.ds(cbase(0), CHUNK)], tgt0)
    pltpu.async_copy(src_hbm.at[pl.ds(cbase(1), CHUNK)], src1, isem1)
    pltpu.async_copy(tgt_hbm.at[pl.ds(cbase(1), CHUNK)], tgt1, isem1)
    pltpu.async_copy(a_hbm.at[src0], ar0, gsem0)
    pltpu.async_copy(b_hbm.at[tgt0], br0, gsem0)
    pltpu.async_copy(c_hbm.at[pl.ds(cbase(0), CHUNK)], cr0, gsem0)

    def _iter(i, _):
        for k in (0, 1):
            srcv, tgtv, stgt, ar, br, cr, gsem, isem, ssem = sets[k]
            (osrc, otgt, ostgt, oar, obr, ocr,
             ogsem, oisem, ossem) = sets[1 - k]
            j = i * 2 + k
            pltpu.make_async_copy(a_hbm.at[srcv], ar, gsem).wait()
            pltpu.make_async_copy(b_hbm.at[tgtv], br, gsem).wait()
            pltpu.make_async_copy(c_hbm.at[pl.ds(cbase(j), CHUNK)], cr,
                                  gsem).wait()

            def _drain_other_scatter():
                pltpu.make_async_copy(oar, hsum_sh.at[ostgt], ossem).wait()
                pltpu.make_async_copy(degrow, deg_sh.at[ostgt], ossem).wait()

            def _fire_next():
                pltpu.make_async_copy(
                    src_hbm.at[pl.ds(cbase(j + 1), CHUNK)], osrc, oisem).wait()
                pltpu.make_async_copy(
                    tgt_hbm.at[pl.ds(cbase(j + 1), CHUNK)], otgt, oisem).wait()
                pltpu.async_copy(a_hbm.at[osrc], oar, ogsem)
                pltpu.async_copy(b_hbm.at[otgt], obr, ogsem)
                pltpu.async_copy(c_hbm.at[pl.ds(cbase(j + 1), CHUNK)],
                                 ocr, ogsem)

            if k == 0:
                pl.when(i > 0)(_drain_other_scatter)
                _fire_next()
            else:
                def _drain_and_fire():
                    _drain_other_scatter()
                    _fire_next()
                pl.when(i < NITER - 1)(_drain_and_fire)

            @plsc.parallel_loop(0, CHUNK, unroll=4)
            def _row(r):
                for cc in range(HIDDEN // LANES):
                    sl = pl.ds(cc * LANES, LANES)
                    ar[r, sl] = jnp.maximum(ar[r, sl] + br[r, sl] + cr[r, sl],
                                            jnp.float32(0.0))

            stgt[pl.ds(0, LANES)] = tgtv[pl.ds(0, LANES)]
            stgt[pl.ds(LANES, LANES)] = tgtv[pl.ds(LANES, LANES)]
            stgt[pl.ds(CHUNK - LANES, LANES)] = tgtv[pl.ds(CHUNK - LANES,
                                                           LANES)]
            pltpu.async_copy(ar, hsum_sh.at[stgt], ssem, add=True)
            pltpu.async_copy(degrow, deg_sh.at[stgt], ssem, add=True)

            @pl.when(i < NITER - 1)
            def _():
                pltpu.async_copy(src_hbm.at[pl.ds(cbase(j + 2), CHUNK)],
                                 srcv, isem)
                pltpu.async_copy(tgt_hbm.at[pl.ds(cbase(j + 2), CHUNK)],
                                 tgtv, isem)
        return 0

    lax.fori_loop(0, NITER, _iter, 0)

    pltpu.make_async_copy(ar0, hsum_sh.at[stgt0], ssem0).wait()
    pltpu.make_async_copy(degrow, deg_sh.at[stgt0], ssem0).wait()
    pltpu.make_async_copy(ar1, hsum_sh.at[stgt1], ssem1).wait()
    pltpu.make_async_copy(degrow, deg_sh.at[stgt1], ssem1).wait()

    plsc.subcore_barrier()

    pltpu.sync_copy(hsum_sh.at[pl.ds(row0, SUB_ROWS)],
                    hsum_out.at[cid, pl.ds(row0, SUB_ROWS)])
    pltpu.sync_copy(deg_sh.at[pl.ds(row0, SUB_ROWS)],
                    deg_out.at[cid, pl.ds(row0, SUB_ROWS)])

    @pl.when(sid == NS - 1)
    def _():
        tail0 = NS * SUB_ROWS
        pltpu.sync_copy(hsum_sh.at[pl.ds(tail0, TAIL_ROWS)],
                        hsum_out.at[cid, pl.ds(tail0, TAIL_ROWS)])
        pltpu.sync_copy(deg_sh.at[pl.ds(tail0, TAIL_ROWS)],
                        deg_out.at[cid, pl.ds(tail0, TAIL_ROWS)])


UPD_GRID = 10
U_BLK = N_NODES // UPD_GRID


def _update_body(nf_ref, p_ref, d_ref, w2t_ref, b2_ref, u1n_ref, u1h_ref,
                 bu1_ref, u2t_ref, bu2_ref, o_ref):
    hsum = p_ref[0] + p_ref[1]
    deg = d_ref[0, :, 0:1] + d_ref[1, :, 0:1]
    agg = (jnp.dot(hsum, w2t_ref[...], preferred_element_type=jnp.float32)
           + deg * b2_ref[...])
    nf = nf_ref[...]
    u = jnp.maximum(
        jnp.dot(nf, u1n_ref[...], preferred_element_type=jnp.float32)
        + jnp.dot(agg, u1h_ref[...], preferred_element_type=jnp.float32)
        + bu1_ref[...], jnp.float32(0.0))
    o_ref[...] = (nf + jnp.dot(u, u2t_ref[...],
                               preferred_element_type=jnp.float32)
                  + bu2_ref[...])


_update = pl.pallas_call(
    _update_body,
    grid=(UPD_GRID,),
    in_specs=[
        pl.BlockSpec((U_BLK, NODE_DIM), lambda i: (i, 0)),
        pl.BlockSpec((NC, U_BLK, HIDDEN), lambda i: (0, i, 0)),
        pl.BlockSpec((NC, U_BLK, DEG_W), lambda i: (0, i, 0)),
        pl.BlockSpec((HIDDEN, HIDDEN), lambda i: (0, 0)),
        pl.BlockSpec((1, HIDDEN), lambda i: (0, 0)),
        pl.BlockSpec((NODE_DIM, HIDDEN), lambda i: (0, 0)),
        pl.BlockSpec((HIDDEN, HIDDEN), lambda i: (0, 0)),
        pl.BlockSpec((1, HIDDEN), lambda i: (0, 0)),
        pl.BlockSpec((HIDDEN, NODE_DIM), lambda i: (0, 0)),
        pl.BlockSpec((1, NODE_DIM), lambda i: (0, 0)),
    ],
    out_specs=pl.BlockSpec((U_BLK, NODE_DIM), lambda i: (i, 0)),
    out_shape=jax.ShapeDtypeStruct((N_NODES, NODE_DIM), jnp.float32),
)


def kernel(node_feats, edge_feats, edge_index, W1, b1, W2, b2, U1, bu1, U2, bu2):
    ei = edge_index.astype(jnp.int32)
    src = ei[0]
    tgt = ei[1]
    w1s = W1[:, :NODE_DIM].T
    w1t = W1[:, NODE_DIM:2 * NODE_DIM].T
    w1e = W1[:, 2 * NODE_DIM:].T
    a_tab, b_tab, c_rows = _prep(node_feats, edge_feats, w1s, w1t, w1e,
                                 b1.reshape(1, HIDDEN))
    hsum, deg = _sc_aggregate(a_tab, b_tab, c_rows, src, tgt)
    return _update(node_feats, hsum, deg, W2.T, b2.reshape(1, HIDDEN),
                   U1[:, :NODE_DIM].T, U1[:, NODE_DIM:].T,
                   bu1.reshape(1, HIDDEN), U2.T, bu2.reshape(1, NODE_DIM))


# speedup vs baseline: 4.9986x; 1.0721x over previous
import functools

import jax
import jax.numpy as jnp
from jax import lax
from jax.experimental import pallas as pl
from jax.experimental.pallas import tpu as pltpu
from jax.experimental.pallas import tpu_sc as plsc

N_NODES = 10000
NODE_DIM = 128
HIDDEN = 128
EDGE_DIM = 16
N_EDGES = 320000

NC, NS = 2, 16
NW = NC * NS
CHUNK = 40
W_CHUNKS = N_EDGES // (NW * CHUNK)
NITER = W_CHUNKS // 2
SUB_ROWS = 624
TAIL_ROWS = N_NODES - NS * SUB_ROWS
DEG_W = 16
LANES = 16


PREP_GRID = 50
E_BLK = N_EDGES // PREP_GRID
N_BLK = N_NODES // PREP_GRID


def _prep_body(nf_ref, ef_ref, w1s_ref, w1t_ref, w1e_ref, b1_ref,
               a_ref, b_ref, c_ref):
    nf = nf_ref[...]
    a_ref[...] = jnp.dot(nf, w1s_ref[...], preferred_element_type=jnp.float32)
    b_ref[...] = jnp.dot(nf, w1t_ref[...], preferred_element_type=jnp.float32)
    c_ref[...] = (jnp.dot(ef_ref[...], w1e_ref[...],
                          preferred_element_type=jnp.float32) + b1_ref[...])


_prep = pl.pallas_call(
    _prep_body,
    grid=(PREP_GRID,),
    in_specs=[
        pl.BlockSpec((N_BLK, NODE_DIM), lambda i: (i, 0)),
        pl.BlockSpec((E_BLK, EDGE_DIM), lambda i: (i, 0)),
        pl.BlockSpec((NODE_DIM, HIDDEN), lambda i: (0, 0)),
        pl.BlockSpec((NODE_DIM, HIDDEN), lambda i: (0, 0)),
        pl.BlockSpec((EDGE_DIM, HIDDEN), lambda i: (0, 0)),
        pl.BlockSpec((1, HIDDEN), lambda i: (0, 0)),
    ],
    out_specs=[
        pl.BlockSpec((N_BLK, HIDDEN), lambda i: (i, 0)),
        pl.BlockSpec((N_BLK, HIDDEN), lambda i: (i, 0)),
        pl.BlockSpec((E_BLK, HIDDEN), lambda i: (i, 0)),
    ],
    out_shape=[
        jax.ShapeDtypeStruct((N_NODES, HIDDEN), jnp.float32),
        jax.ShapeDtypeStruct((N_NODES, HIDDEN), jnp.float32),
        jax.ShapeDtypeStruct((N_EDGES, HIDDEN), jnp.float32),
    ],
)


_sc_mesh = plsc.VectorSubcoreMesh(core_axis_name="c", subcore_axis_name="s")


@functools.partial(
    pl.kernel,
    mesh=_sc_mesh,
    compiler_params=pltpu.CompilerParams(use_tc_tiling_on_sc=False),
    out_type=[
        pltpu.HBM((NC, N_NODES, HIDDEN), jnp.float32),
        pltpu.HBM((NC, N_NODES, DEG_W), jnp.float32),
    ],
    scratch_types=[
        pltpu.VMEM((CHUNK,), jnp.int32),
        pltpu.VMEM((CHUNK,), jnp.int32),
        pltpu.VMEM((CHUNK,), jnp.int32),
        pltpu.VMEM((CHUNK,), jnp.int32),
        pltpu.VMEM((CHUNK,), jnp.int32),
        pltpu.VMEM((CHUNK,), jnp.int32),
        pltpu.VMEM((CHUNK, HIDDEN), jnp.float32),
        pltpu.VMEM((CHUNK, HIDDEN), jnp.float32),
        pltpu.VMEM((CHUNK, HIDDEN), jnp.float32),
        pltpu.VMEM((CHUNK, HIDDEN), jnp.float32),
        pltpu.VMEM((CHUNK, HIDDEN), jnp.float32),
        pltpu.VMEM((CHUNK, HIDDEN), jnp.float32),
        pltpu.VMEM((CHUNK, DEG_W), jnp.float32),
        pltpu.VMEM_SHARED((N_NODES, HIDDEN), jnp.float32),
        pltpu.VMEM_SHARED((N_NODES, DEG_W), jnp.float32),
        pltpu.SemaphoreType.DMA,
        pltpu.SemaphoreType.DMA,
        pltpu.SemaphoreType.DMA,
        pltpu.SemaphoreType.DMA,
        pltpu.SemaphoreType.DMA,
        pltpu.SemaphoreType.DMA,
    ],
)
def _sc_aggregate(a_hbm, b_hbm, c_hbm, src_hbm, tgt_hbm,
                  hsum_out, deg_out,
                  src0, tgt0, src1, tgt1, stgt0, stgt1,
                  ar0, br0, cr0, ar1, br1, cr1,
                  degrow, hsum_sh, deg_sh,
                  gsem0, gsem1, isem0, isem1, ssem0, ssem1):
    cid = lax.axis_index("c")
    sid = lax.axis_index("s")
    wid = sid * NC + cid
    sets = ((src0, tgt0, stgt0, ar0, br0, cr0, gsem0, isem0, ssem0),
            (src1, tgt1, stgt1, ar1, br1, cr1, gsem1, isem1, ssem1))

    lanes = lax.iota(jnp.int32, LANES)
    zero16 = jnp.zeros((LANES,), jnp.float32)
    one0 = jnp.where(lanes == 0, jnp.float32(1.0), jnp.float32(0.0))

    def _fill_zeros(r, _):
        for cc in range(HIDDEN // LANES):
            ar0[r, pl.ds(cc * LANES, LANES)] = zero16
        degrow[r, pl.ds(0, LANES)] = zero16
        return 0

    lax.fori_loop(0, CHUNK, _fill_zeros, 0)

    row0 = sid * SUB_ROWS
    for z in range(SUB_ROWS // CHUNK):
        pltpu.sync_copy(ar0, hsum_sh.at[pl.ds(row0 + z * CHUNK, CHUNK)])
        pltpu.sync_copy(degrow, deg_sh.at[pl.ds(row0 + z * CHUNK, CHUNK)])
    _rem = SUB_ROWS - (SUB_ROWS // CHUNK) * CHUNK
    pltpu.sync_copy(ar0.at[pl.ds(0, _rem)],
                    hsum_sh.at[pl.ds(row0 + SUB_ROWS - _rem, _rem)])
    pltpu.sync_copy(degrow.at[pl.ds(0, _rem)],
                    deg_sh.at[pl.ds(row0 + SUB_ROWS - _rem, _rem)])

    @pl.when(sid == NS - 1)
    def _():
        tail0 = NS * SUB_ROWS
        pltpu.sync_copy(ar0.at[pl.ds(0, TAIL_ROWS)],
                        hsum_sh.at[pl.ds(tail0, TAIL_ROWS)])
        pltpu.sync_copy(degrow.at[pl.ds(0, TAIL_ROWS)],
                        deg_sh.at[pl.ds(tail0, TAIL_ROWS)])

    def _fill_deg(r, _):
        degrow[r, pl.ds(0, LANES)] = one0
        return 0

    lax.fori_loop(0, CHUNK, _fill_deg, 0)

    plsc.subcore_barrier()

    def cbase(j):
        return (wid * W_CHUNKS + j) * CHUNK

    pltpu.sync_copy(src_hbm.at[pl.ds(cbase(0), CHUNK)], src0)
    pltpu.sync_copy(tgt_hbm.at[pl.ds(cbase(0), CHUNK)], tgt0)
    pltpu.async_copy(src_hbm.at[pl.ds(cbase(1), CHUNK)], src1, isem1)
    pltpu.async_copy(tgt_hbm.at[pl.ds(cbase(1), CHUNK)], tgt1, isem1)
    pltpu.async_copy(a_hbm.at[src0], ar0, gsem0)
    pltpu.async_copy(b_hbm.at[tgt0], br0, gsem0)
    pltpu.async_copy(c_hbm.at[pl.ds(cbase(0), CHUNK)], cr0, gsem0)

    def _iter(i, _):
        for k in (0, 1):
            srcv, tgtv, stgt, ar, br, cr, gsem, isem, ssem = sets[k]
            (osrc, otgt, ostgt, oar, obr, ocr,
             ogsem, oisem, ossem) = sets[1 - k]
            j = i * 2 + k
            pltpu.make_async_copy(a_hbm.at[srcv], ar, gsem).wait()
            pltpu.make_async_copy(b_hbm.at[tgtv], br, gsem).wait()
            pltpu.make_async_copy(c_hbm.at[pl.ds(cbase(j), CHUNK)], cr,
                                  gsem).wait()

            def _drain_other_scatter():
                pltpu.make_async_copy(oar, hsum_sh.at[ostgt], ossem).wait()
                pltpu.make_async_copy(degrow, deg_sh.at[ostgt], ossem).wait()

            def _fire_next():
                pltpu.make_async_copy(
                    src_hbm.at[pl.ds(cbase(j + 1), CHUNK)], osrc, oisem).wait()
                pltpu.make_async_copy(
                    tgt_hbm.at[pl.ds(cbase(j + 1), CHUNK)], otgt, oisem).wait()
                pltpu.async_copy(a_hbm.at[osrc], oar, ogsem)
                pltpu.async_copy(b_hbm.at[otgt], obr, ogsem)
                pltpu.async_copy(c_hbm.at[pl.ds(cbase(j + 1), CHUNK)],
                                 ocr, ogsem)

            if k == 0:
                pl.when(i > 0)(_drain_other_scatter)
                _fire_next()
            else:
                def _drain_and_fire():
                    _drain_other_scatter()
                    _fire_next()
                pl.when(i < NITER - 1)(_drain_and_fire)

            @plsc.parallel_loop(0, CHUNK, unroll=4)
            def _row(r):
                for cc in range(HIDDEN // LANES):
                    sl = pl.ds(cc * LANES, LANES)
                    ar[r, sl] = jnp.maximum(ar[r, sl] + br[r, sl] + cr[r, sl],
                                            jnp.float32(0.0))

            stgt[pl.ds(0, LANES)] = tgtv[pl.ds(0, LANES)]
            stgt[pl.ds(LANES, LANES)] = tgtv[pl.ds(LANES, LANES)]
            stgt[pl.ds(CHUNK - LANES, LANES)] = tgtv[pl.ds(CHUNK - LANES,
                                                           LANES)]
            pltpu.async_copy(ar, hsum_sh.at[stgt], ssem, add=True)
            pltpu.async_copy(degrow, deg_sh.at[stgt], ssem, add=True)

            @pl.when(i < NITER - 1)
            def _():
                pltpu.async_copy(src_hbm.at[pl.ds(cbase(j + 2), CHUNK)],
                                 srcv, isem)
                pltpu.async_copy(tgt_hbm.at[pl.ds(cbase(j + 2), CHUNK)],
                                 tgtv, isem)
        return 0

    lax.fori_loop(0, NITER, _iter, 0)

    pltpu.make_async_copy(ar0, hsum_sh.at[stgt0], ssem0).wait()
    pltpu.make_async_copy(degrow, deg_sh.at[stgt0], ssem0).wait()
    pltpu.make_async_copy(ar1, hsum_sh.at[stgt1], ssem1).wait()
    pltpu.make_async_copy(degrow, deg_sh.at[stgt1], ssem1).wait()

    plsc.subcore_barrier()

    pltpu.sync_copy(hsum_sh.at[pl.ds(row0, SUB_ROWS)],
                    hsum_out.at[cid, pl.ds(row0, SUB_ROWS)])
    pltpu.sync_copy(deg_sh.at[pl.ds(row0, SUB_ROWS)],
                    deg_out.at[cid, pl.ds(row0, SUB_ROWS)])

    @pl.when(sid == NS - 1)
    def _():
        tail0 = NS * SUB_ROWS
        pltpu.sync_copy(hsum_sh.at[pl.ds(tail0, TAIL_ROWS)],
                        hsum_out.at[cid, pl.ds(tail0, TAIL_ROWS)])
        pltpu.sync_copy(deg_sh.at[pl.ds(tail0, TAIL_ROWS)],
                        deg_out.at[cid, pl.ds(tail0, TAIL_ROWS)])


UPD_GRID = 10
U_BLK = N_NODES // UPD_GRID


def _update_body(nf_ref, p_ref, d_ref, w2t_ref, b2_ref, u1n_ref, u1h_ref,
                 bu1_ref, u2t_ref, bu2_ref, o_ref):
    hsum = p_ref[0] + p_ref[1]
    deg = d_ref[0, :, 0:1] + d_ref[1, :, 0:1]
    agg = (jnp.dot(hsum, w2t_ref[...], preferred_element_type=jnp.float32)
           + deg * b2_ref[...])
    nf = nf_ref[...]
    u = jnp.maximum(
        jnp.dot(nf, u1n_ref[...], preferred_element_type=jnp.float32)
        + jnp.dot(agg, u1h_ref[...], preferred_element_type=jnp.float32)
        + bu1_ref[...], jnp.float32(0.0))
    o_ref[...] = (nf + jnp.dot(u, u2t_ref[...],
                               preferred_element_type=jnp.float32)
                  + bu2_ref[...])


_update = pl.pallas_call(
    _update_body,
    grid=(UPD_GRID,),
    in_specs=[
        pl.BlockSpec((U_BLK, NODE_DIM), lambda i: (i, 0)),
        pl.BlockSpec((NC, U_BLK, HIDDEN), lambda i: (0, i, 0)),
        pl.BlockSpec((NC, U_BLK, DEG_W), lambda i: (0, i, 0)),
        pl.BlockSpec((HIDDEN, HIDDEN), lambda i: (0, 0)),
        pl.BlockSpec((1, HIDDEN), lambda i: (0, 0)),
        pl.BlockSpec((NODE_DIM, HIDDEN), lambda i: (0, 0)),
        pl.BlockSpec((HIDDEN, HIDDEN), lambda i: (0, 0)),
        pl.BlockSpec((1, HIDDEN), lambda i: (0, 0)),
        pl.BlockSpec((HIDDEN, NODE_DIM), lambda i: (0, 0)),
        pl.BlockSpec((1, NODE_DIM), lambda i: (0, 0)),
    ],
    out_specs=pl.BlockSpec((U_BLK, NODE_DIM), lambda i: (i, 0)),
    out_shape=jax.ShapeDtypeStruct((N_NODES, NODE_DIM), jnp.float32),
)


def kernel(node_feats, edge_feats, edge_index, W1, b1, W2, b2, U1, bu1, U2, bu2):
    ei = edge_index.astype(jnp.int32)
    src = ei[0]
    tgt = ei[1]
    w1s = W1[:, :NODE_DIM].T
    w1t = W1[:, NODE_DIM:2 * NODE_DIM].T
    w1e = W1[:, 2 * NODE_DIM:].T
    a_tab, b_tab, c_rows = _prep(node_feats, edge_feats, w1s, w1t, w1e,
                                 b1.reshape(1, HIDDEN))
    hsum, deg = _sc_aggregate(a_tab, b_tab, c_rows, src, tgt)
    return _update(node_feats, hsum, deg, W2.T, b2.reshape(1, HIDDEN),
                   U1[:, :NODE_DIM].T, U1[:, NODE_DIM:].T,
                   bu1.reshape(1, HIDDEN), U2.T, bu2.reshape(1, NODE_DIM))


# speedup vs baseline: 5.0173x; 1.0037x over previous
import functools

import jax
import jax.numpy as jnp
from jax import lax
from jax.experimental import pallas as pl
from jax.experimental.pallas import tpu as pltpu
from jax.experimental.pallas import tpu_sc as plsc

N_NODES = 10000
NODE_DIM = 128
HIDDEN = 128
EDGE_DIM = 16
N_EDGES = 320000

NC, NS = 2, 16
NW = NC * NS
CHUNK = 40
W_CHUNKS = N_EDGES // (NW * CHUNK)
NITER = W_CHUNKS // 2
SUB_ROWS = 624
TAIL_ROWS = N_NODES - NS * SUB_ROWS
DEG_W = 16
LANES = 16


PREP_GRID = 25
E_BLK = N_EDGES // PREP_GRID
N_BLK = N_NODES // PREP_GRID


def _prep_body(nf_ref, ef_ref, w1s_ref, w1t_ref, w1e_ref, b1_ref,
               a_ref, b_ref, c_ref):
    nf = nf_ref[...]
    a_ref[...] = jnp.dot(nf, w1s_ref[...], preferred_element_type=jnp.float32)
    b_ref[...] = jnp.dot(nf, w1t_ref[...], preferred_element_type=jnp.float32)
    c_ref[...] = (jnp.dot(ef_ref[...], w1e_ref[...],
                          preferred_element_type=jnp.float32) + b1_ref[...])


_prep = pl.pallas_call(
    _prep_body,
    grid=(PREP_GRID,),
    in_specs=[
        pl.BlockSpec((N_BLK, NODE_DIM), lambda i: (i, 0)),
        pl.BlockSpec((E_BLK, EDGE_DIM), lambda i: (i, 0)),
        pl.BlockSpec((NODE_DIM, HIDDEN), lambda i: (0, 0)),
        pl.BlockSpec((NODE_DIM, HIDDEN), lambda i: (0, 0)),
        pl.BlockSpec((EDGE_DIM, HIDDEN), lambda i: (0, 0)),
        pl.BlockSpec((1, HIDDEN), lambda i: (0, 0)),
    ],
    out_specs=[
        pl.BlockSpec((N_BLK, HIDDEN), lambda i: (i, 0)),
        pl.BlockSpec((N_BLK, HIDDEN), lambda i: (i, 0)),
        pl.BlockSpec((E_BLK, HIDDEN), lambda i: (i, 0)),
    ],
    out_shape=[
        jax.ShapeDtypeStruct((N_NODES, HIDDEN), jnp.float32),
        jax.ShapeDtypeStruct((N_NODES, HIDDEN), jnp.float32),
        jax.ShapeDtypeStruct((N_EDGES, HIDDEN), jnp.float32),
    ],
)


_sc_mesh = plsc.VectorSubcoreMesh(core_axis_name="c", subcore_axis_name="s")


@functools.partial(
    pl.kernel,
    mesh=_sc_mesh,
    compiler_params=pltpu.CompilerParams(use_tc_tiling_on_sc=False),
    out_type=[
        pltpu.HBM((NC, N_NODES, HIDDEN), jnp.float32),
        pltpu.HBM((NC, N_NODES, DEG_W), jnp.float32),
    ],
    scratch_types=[
        pltpu.VMEM((CHUNK,), jnp.int32),
        pltpu.VMEM((CHUNK,), jnp.int32),
        pltpu.VMEM((CHUNK,), jnp.int32),
        pltpu.VMEM((CHUNK,), jnp.int32),
        pltpu.VMEM((CHUNK,), jnp.int32),
        pltpu.VMEM((CHUNK,), jnp.int32),
        pltpu.VMEM((CHUNK, HIDDEN), jnp.float32),
        pltpu.VMEM((CHUNK, HIDDEN), jnp.float32),
        pltpu.VMEM((CHUNK, HIDDEN), jnp.float32),
        pltpu.VMEM((CHUNK, HIDDEN), jnp.float32),
        pltpu.VMEM((CHUNK, HIDDEN), jnp.float32),
        pltpu.VMEM((CHUNK, HIDDEN), jnp.float32),
        pltpu.VMEM((CHUNK, DEG_W), jnp.float32),
        pltpu.VMEM_SHARED((N_NODES, HIDDEN), jnp.float32),
        pltpu.VMEM_SHARED((N_NODES, DEG_W), jnp.float32),
        pltpu.SemaphoreType.DMA,
        pltpu.SemaphoreType.DMA,
        pltpu.SemaphoreType.DMA,
        pltpu.SemaphoreType.DMA,
        pltpu.SemaphoreType.DMA,
        pltpu.SemaphoreType.DMA,
    ],
)
def _sc_aggregate(a_hbm, b_hbm, c_hbm, src_hbm, tgt_hbm,
                  hsum_out, deg_out,
                  src0, tgt0, src1, tgt1, stgt0, stgt1,
                  ar0, br0, cr0, ar1, br1, cr1,
                  degrow, hsum_sh, deg_sh,
                  gsem0, gsem1, isem0, isem1, ssem0, ssem1):
    cid = lax.axis_index("c")
    sid = lax.axis_index("s")
    wid = sid * NC + cid
    sets = ((src0, tgt0, stgt0, ar0, br0, cr0, gsem0, isem0, ssem0),
            (src1, tgt1, stgt1, ar1, br1, cr1, gsem1, isem1, ssem1))

    lanes = lax.iota(jnp.int32, LANES)
    zero16 = jnp.zeros((LANES,), jnp.float32)
    one0 = jnp.where(lanes == 0, jnp.float32(1.0), jnp.float32(0.0))

    def _fill_zeros(r, _):
        for cc in range(HIDDEN // LANES):
            ar0[r, pl.ds(cc * LANES, LANES)] = zero16
        degrow[r, pl.ds(0, LANES)] = zero16
        return 0

    lax.fori_loop(0, CHUNK, _fill_zeros, 0)

    row0 = sid * SUB_ROWS
    for z in range(SUB_ROWS // CHUNK):
        pltpu.sync_copy(ar0, hsum_sh.at[pl.ds(row0 + z * CHUNK, CHUNK)])
        pltpu.sync_copy(degrow, deg_sh.at[pl.ds(row0 + z * CHUNK, CHUNK)])
    _rem = SUB_ROWS - (SUB_ROWS // CHUNK) * CHUNK
    pltpu.sync_copy(ar0.at[pl.ds(0, _rem)],
                    hsum_sh.at[pl.ds(row0 + SUB_ROWS - _rem, _rem)])
    pltpu.sync_copy(degrow.at[pl.ds(0, _rem)],
                    deg_sh.at[pl.ds(row0 + SUB_ROWS - _rem, _rem)])

    @pl.when(sid == NS - 1)
    def _():
        tail0 = NS * SUB_ROWS
        pltpu.sync_copy(ar0.at[pl.ds(0, TAIL_ROWS)],
                        hsum_sh.at[pl.ds(tail0, TAIL_ROWS)])
        pltpu.sync_copy(degrow.at[pl.ds(0, TAIL_ROWS)],
                        deg_sh.at[pl.ds(tail0, TAIL_ROWS)])

    def _fill_deg(r, _):
        degrow[r, pl.ds(0, LANES)] = one0
        return 0

    lax.fori_loop(0, CHUNK, _fill_deg, 0)

    plsc.subcore_barrier()

    def cbase(j):
        return (wid * W_CHUNKS + j) * CHUNK

    pltpu.sync_copy(src_hbm.at[pl.ds(cbase(0), CHUNK)], src0)
    pltpu.sync_copy(tgt_hbm.at[pl.ds(cbase(0), CHUNK)], tgt0)
    pltpu.async_copy(src_hbm.at[pl.ds(cbase(1), CHUNK)], src1, isem1)
    pltpu.async_copy(tgt_hbm.at[pl.ds(cbase(1), CHUNK)], tgt1, isem1)
    pltpu.async_copy(a_hbm.at[src0], ar0, gsem0)
    pltpu.async_copy(b_hbm.at[tgt0], br0, gsem0)
    pltpu.async_copy(c_hbm.at[pl.ds(cbase(0), CHUNK)], cr0, gsem0)

    def _iter(i, _):
        for k in (0, 1):
            srcv, tgtv, stgt, ar, br, cr, gsem, isem, ssem = sets[k]
            (osrc, otgt, ostgt, oar, obr, ocr,
             ogsem, oisem, ossem) = sets[1 - k]
            j = i * 2 + k
            pltpu.make_async_copy(a_hbm.at[srcv], ar, gsem).wait()
            pltpu.make_async_copy(b_hbm.at[tgtv], br, gsem).wait()
            pltpu.make_async_copy(c_hbm.at[pl.ds(cbase(j), CHUNK)], cr,
                                  gsem).wait()

            def _drain_other_scatter():
                pltpu.make_async_copy(oar, hsum_sh.at[ostgt], ossem).wait()
                pltpu.make_async_copy(degrow, deg_sh.at[ostgt], ossem).wait()

            def _fire_next():
                pltpu.make_async_copy(
                    src_hbm.at[pl.ds(cbase(j + 1), CHUNK)], osrc, oisem).wait()
                pltpu.make_async_copy(
                    tgt_hbm.at[pl.ds(cbase(j + 1), CHUNK)], otgt, oisem).wait()
                pltpu.async_copy(a_hbm.at[osrc], oar, ogsem)
                pltpu.async_copy(b_hbm.at[otgt], obr, ogsem)
                pltpu.async_copy(c_hbm.at[pl.ds(cbase(j + 1), CHUNK)],
                                 ocr, ogsem)

            if k == 0:
                pl.when(i > 0)(_drain_other_scatter)
                _fire_next()
            else:
                def _drain_and_fire():
                    _drain_other_scatter()
                    _fire_next()
                pl.when(i < NITER - 1)(_drain_and_fire)

            @plsc.parallel_loop(0, CHUNK, unroll=4)
            def _row(r):
                for cc in range(HIDDEN // LANES):
                    sl = pl.ds(cc * LANES, LANES)
                    ar[r, sl] = jnp.maximum(ar[r, sl] + br[r, sl] + cr[r, sl],
                                            jnp.float32(0.0))

            stgt[pl.ds(0, LANES)] = tgtv[pl.ds(0, LANES)]
            stgt[pl.ds(LANES, LANES)] = tgtv[pl.ds(LANES, LANES)]
            stgt[pl.ds(CHUNK - LANES, LANES)] = tgtv[pl.ds(CHUNK - LANES,
                                                           LANES)]
            pltpu.async_copy(ar, hsum_sh.at[stgt], ssem, add=True)
            pltpu.async_copy(degrow, deg_sh.at[stgt], ssem, add=True)

            @pl.when(i < NITER - 1)
            def _():
                pltpu.async_copy(src_hbm.at[pl.ds(cbase(j + 2), CHUNK)],
                                 srcv, isem)
                pltpu.async_copy(tgt_hbm.at[pl.ds(cbase(j + 2), CHUNK)],
                                 tgtv, isem)
        return 0

    lax.fori_loop(0, NITER, _iter, 0)

    pltpu.make_async_copy(ar0, hsum_sh.at[stgt0], ssem0).wait()
    pltpu.make_async_copy(degrow, deg_sh.at[stgt0], ssem0).wait()
    pltpu.make_async_copy(ar1, hsum_sh.at[stgt1], ssem1).wait()
    pltpu.make_async_copy(degrow, deg_sh.at[stgt1], ssem1).wait()

    plsc.subcore_barrier()

    pltpu.sync_copy(hsum_sh.at[pl.ds(row0, SUB_ROWS)],
                    hsum_out.at[cid, pl.ds(row0, SUB_ROWS)])
    pltpu.sync_copy(deg_sh.at[pl.ds(row0, SUB_ROWS)],
                    deg_out.at[cid, pl.ds(row0, SUB_ROWS)])

    @pl.when(sid == NS - 1)
    def _():
        tail0 = NS * SUB_ROWS
        pltpu.sync_copy(hsum_sh.at[pl.ds(tail0, TAIL_ROWS)],
                        hsum_out.at[cid, pl.ds(tail0, TAIL_ROWS)])
        pltpu.sync_copy(deg_sh.at[pl.ds(tail0, TAIL_ROWS)],
                        deg_out.at[cid, pl.ds(tail0, TAIL_ROWS)])


UPD_GRID = 10
U_BLK = N_NODES // UPD_GRID


def _update_body(nf_ref, p_ref, d_ref, w2t_ref, b2_ref, u1n_ref, u1h_ref,
                 bu1_ref, u2t_ref, bu2_ref, o_ref):
    hsum = p_ref[0] + p_ref[1]
    deg = d_ref[0, :, 0:1] + d_ref[1, :, 0:1]
    agg = (jnp.dot(hsum, w2t_ref[...], preferred_element_type=jnp.float32)
           + deg * b2_ref[...])
    nf = nf_ref[...]
    u = jnp.maximum(
        jnp.dot(nf, u1n_ref[...], preferred_element_type=jnp.float32)
        + jnp.dot(agg, u1h_ref[...], preferred_element_type=jnp.float32)
        + bu1_ref[...], jnp.float32(0.0))
    o_ref[...] = (nf + jnp.dot(u, u2t_ref[...],
                               preferred_element_type=jnp.float32)
                  + bu2_ref[...])


_update = pl.pallas_call(
    _update_body,
    grid=(UPD_GRID,),
    in_specs=[
        pl.BlockSpec((U_BLK, NODE_DIM), lambda i: (i, 0)),
        pl.BlockSpec((NC, U_BLK, HIDDEN), lambda i: (0, i, 0)),
        pl.BlockSpec((NC, U_BLK, DEG_W), lambda i: (0, i, 0)),
        pl.BlockSpec((HIDDEN, HIDDEN), lambda i: (0, 0)),
        pl.BlockSpec((1, HIDDEN), lambda i: (0, 0)),
        pl.BlockSpec((NODE_DIM, HIDDEN), lambda i: (0, 0)),
        pl.BlockSpec((HIDDEN, HIDDEN), lambda i: (0, 0)),
        pl.BlockSpec((1, HIDDEN), lambda i: (0, 0)),
        pl.BlockSpec((HIDDEN, NODE_DIM), lambda i: (0, 0)),
        pl.BlockSpec((1, NODE_DIM), lambda i: (0, 0)),
    ],
    out_specs=pl.BlockSpec((U_BLK, NODE_DIM), lambda i: (i, 0)),
    out_shape=jax.ShapeDtypeStruct((N_NODES, NODE_DIM), jnp.float32),
)


def kernel(node_feats, edge_feats, edge_index, W1, b1, W2, b2, U1, bu1, U2, bu2):
    ei = edge_index.astype(jnp.int32)
    src = ei[0]
    tgt = ei[1]
    w1s = W1[:, :NODE_DIM].T
    w1t = W1[:, NODE_DIM:2 * NODE_DIM].T
    w1e = W1[:, 2 * NODE_DIM:].T
    a_tab, b_tab, c_rows = _prep(node_feats, edge_feats, w1s, w1t, w1e,
                                 b1.reshape(1, HIDDEN))
    hsum, deg = _sc_aggregate(a_tab, b_tab, c_rows, src, tgt)
    return _update(node_feats, hsum, deg, W2.T, b2.reshape(1, HIDDEN),
                   U1[:, :NODE_DIM].T, U1[:, NODE_DIM:].T,
                   bu1.reshape(1, HIDDEN), U2.T, bu2.reshape(1, NODE_DIM))
